# baseline (speedup 1.0000x reference)
import functools

import jax
import jax.numpy as jnp
from jax import lax
from jax.experimental import pallas as pl
from jax.experimental.pallas import tpu as pltpu
from jax.experimental.pallas import tpu_sc as plsc

ALPHA = 0.2
LANES = 16
CP = 48
NC = 2
NS = 16
NW = NC * NS
CH = 256

_DN = lax.GatherDimensionNumbers(
    offset_dims=(), collapsed_slice_dims=(0,), start_index_map=(0,))


def _take16(vec, i):
    idx = jnp.full((LANES, 1), i, jnp.int32)
    return lax.gather(vec, idx, _DN, (1,),
                      mode=lax.GatherScatterMode.PROMISE_IN_BOUNDS)


def _project_body(h_ref, w_ref, a2_ref, hw_ref, s_ref):
    hW = jnp.dot(h_ref[...], w_ref[...], preferred_element_type=jnp.float32)
    b = hW.shape[0]
    ones = jnp.ones((b, 1), jnp.float32)
    zeros = jnp.zeros((b, CP - hW.shape[1] - 1), jnp.float32)
    hw_ref[...] = jnp.concatenate([hW, ones, zeros], axis=1)
    s_ref[...] = lax.dot_general(
        hW, a2_ref[...], (((1,), (1,)), ((), ())),
        preferred_element_type=jnp.float32)


def _project(h, W, a2):
    n, d = h.shape
    c = W.shape[1]
    blk = 2000
    grid = n // blk
    return pl.pallas_call(
        _project_body,
        grid=(grid,),
        in_specs=[
            pl.BlockSpec((blk, d), lambda i: (i, 0)),
            pl.BlockSpec((d, c), lambda i: (0, 0)),
            pl.BlockSpec((2, c), lambda i: (0, 0)),
        ],
        out_specs=[
            pl.BlockSpec((blk, CP), lambda i: (i, 0)),
            pl.BlockSpec((blk, 2), lambda i: (i, 0)),
        ],
        out_shape=[
            jax.ShapeDtypeStruct((n, CP), jnp.float32),
            jax.ShapeDtypeStruct((n, 2), jnp.float32),
        ],
    )(h, W, a2)


def _sc_edge(hw_pad, s12p, src2d, dst2d, nb):
    n = hw_pad.shape[0]
    s12n = s12p.shape[0]
    rpt = 640
    last_rows = n - (NS - 1) * rpt
    rpt_s = 1256
    last_s = s12n - (NS - 1) * rpt_s

    @functools.partial(
        pl.kernel,
        out_type=jax.ShapeDtypeStruct((NC, n, CP), jnp.float32),
        mesh=plsc.VectorSubcoreMesh(core_axis_name="c", subcore_axis_name="s"),
        compiler_params=pltpu.CompilerParams(needs_layout_passes=False, use_tc_tiling_on_sc=False),
        scratch_types=[
            pltpu.VMEM((s12n,), jnp.float32),
            pltpu.VMEM_SHARED((s12n,), jnp.float32),
            pltpu.VMEM((nb, CH), jnp.int32),
            pltpu.VMEM((nb, CH), jnp.int32),
            pltpu.VMEM((CH, CP), jnp.float32),
            pltpu.VMEM((CH, CP), jnp.float32),
            pltpu.VMEM((CH,), jnp.float32),
            pltpu.VMEM((CH,), jnp.float32),
            pltpu.VMEM_SHARED((n + 8, CP), jnp.float32),
            pltpu.VMEM_SHARED((n, CP), jnp.float32),
            pltpu.SemaphoreType.DMA,
            pltpu.SemaphoreType.DMA,
            pltpu.SemaphoreType.DMA,
            pltpu.SemaphoreType.DMA,
        ],
    )
    def k(hw_hbm, s12_hbm, src_hbm, dst_hbm, out_hbm,
          s12_v, s12_sh, src_v, dst_v, rows_a, rows_b, w_a, w_b,
          acc_s, hw_s, g_a, g_b, s_a, s_b):
        sid = lax.axis_index("s")
        cid = lax.axis_index("c")
        wid = sid * NC + cid

        @pl.when(sid < NS - 1)
        def _():
            pltpu.sync_copy(hw_hbm.at[pl.ds(sid * rpt, rpt)],
                            hw_s.at[pl.ds(sid * rpt, rpt)])
            pltpu.sync_copy(s12_hbm.at[pl.ds(sid * rpt_s, rpt_s)],
                            s12_sh.at[pl.ds(sid * rpt_s, rpt_s)])

        @pl.when(sid == NS - 1)
        def _():
            pltpu.sync_copy(hw_hbm.at[pl.ds((NS - 1) * rpt, last_rows)],
                            hw_s.at[pl.ds((NS - 1) * rpt, last_rows)])
            pltpu.sync_copy(s12_hbm.at[pl.ds((NS - 1) * rpt_s, last_s)],
                            s12_sh.at[pl.ds((NS - 1) * rpt_s, last_s)])

        zv = jnp.zeros((LANES,), jnp.float32)
        for r in range(CH):
            for cc in range(CP // LANES):
                rows_a[r, pl.ds(cc * LANES, LANES)] = zv

        def _zero(base, cnt):
            for b in range(cnt // CH):
                pltpu.sync_copy(rows_a, acc_s.at[pl.ds(base + b * CH, CH)])
            rem = cnt % CH
            if rem:
                pltpu.sync_copy(
                    rows_a.at[pl.ds(0, rem)],
                    acc_s.at[pl.ds(base + (cnt // CH) * CH, rem)])

        @pl.when(sid < NS - 1)
        def _():
            _zero(sid * rpt, rpt)

        @pl.when(sid == NS - 1)
        def _():
            _zero((NS - 1) * rpt, last_rows)

        pltpu.sync_copy(src_hbm.at[pl.ds(wid * nb, nb)], src_v)
        pltpu.sync_copy(dst_hbm.at[pl.ds(wid * nb, nb)], dst_v)

        plsc.subcore_barrier()

        pltpu.sync_copy(s12_sh, s12_v)

        def weights(j):
            ws = []
            for g in range(CH // LANES):
                srcv = src_v[j, pl.ds(g * LANES, LANES)]
                dstv = dst_v[j, pl.ds(g * LANES, LANES)]
                s1 = plsc.load_gather(s12_v, [srcv * 2])
                s2 = plsc.load_gather(s12_v, [dstv * 2 + 1])
                logit = s1 + s2
                lk = jnp.where(logit >= 0.0, logit, logit * ALPHA)
                ws.append(jnp.exp(-lk))
            return ws

        def scale(rows_v, ws):
            for g in range(CH // LANES):
                for i in range(LANES):
                    wi = _take16(ws[g], i)
                    r = g * LANES + i
                    for cc in range(CP // LANES):
                        sl = pl.ds(cc * LANES, LANES)
                        rows_v[r, sl] = rows_v[r, sl] * wi

        def gather(j, buf, sem):
            pltpu.async_copy(hw_s.at[dst_v.at[j]], buf, sem)

        def gather_wait(buf, sem):
            pltpu.make_async_copy(hw_s.at[dst_v.at[0]], buf, sem).wait()

        def scat(j, buf, sem):
            pltpu.async_copy(buf, acc_s.at[src_v.at[j]], sem, add=True)

        def scat_wait(buf, sem):
            pltpu.make_async_copy(buf, acc_s.at[src_v.at[0]], sem).wait()

        def body(i, carry):
            t = 2 * i
            gather(t, rows_a, g_a)
            ws_a = weights(t)
            gather_wait(rows_a, g_a)
            gather(t + 1, rows_b, g_b)
            scale(rows_a, ws_a)
            scat(t, rows_a, s_a)
            ws_b = weights(t + 1)
            gather_wait(rows_b, g_b)
            scale(rows_b, ws_b)
            scat(t + 1, rows_b, s_b)
            scat_wait(rows_a, s_a)
            scat_wait(rows_b, s_b)
            return carry

        lax.fori_loop(0, nb // 2, body, 0)

        plsc.subcore_barrier()

        @pl.when(sid < NS - 1)
        def _():
            pltpu.sync_copy(acc_s.at[pl.ds(sid * rpt, rpt)],
                            out_hbm.at[cid, pl.ds(sid * rpt, rpt)])

        @pl.when(sid == NS - 1)
        def _():
            pltpu.sync_copy(
                acc_s.at[pl.ds((NS - 1) * rpt, last_rows)],
                out_hbm.at[cid, pl.ds((NS - 1) * rpt, last_rows)])

    return k(hw_pad, s12p, src2d, dst2d)


def _finalize_body(acc_ref, out_ref):
    a0 = acc_ref[0]
    a1 = acc_ref[1]
    c = out_ref.shape[1]
    num = a0[:, :c] + a1[:, :c]
    den = a0[:, c:c + 1] + a1[:, c:c + 1]
    hp = num / den
    out_ref[...] = jnp.where(hp > 0.0, hp, jnp.exp(hp) - 1.0)


def _finalize(accum, c):
    n = accum.shape[1]
    blk = 1000
    grid = n // blk
    return pl.pallas_call(
        _finalize_body,
        grid=(grid,),
        in_specs=[pl.BlockSpec((NC, blk, CP), lambda i: (0, i, 0))],
        out_specs=pl.BlockSpec((blk, c), lambda i: (i, 0)),
        out_shape=jax.ShapeDtypeStruct((n, c), jnp.float32),
    )(accum)


def kernel(h, edge_index, W, a):
    n = h.shape[0]
    c = W.shape[1]
    e = edge_index.shape[1]
    hw_pad, s12 = _project(h, W, a.reshape(2, c))
    nch = e // CH
    nb = -(-nch // NW)
    nb = nb + (nb % 2)
    pad = nb * NW - nch
    src2d = jnp.pad(edge_index[0], (0, pad * CH),
                    constant_values=n).reshape(nb * NW, CH)
    dst2d = jnp.pad(edge_index[1], (0, pad * CH)).reshape(nb * NW, CH)
    s12p = jnp.pad(s12.reshape(2 * n), (0, 16))
    accum = _sc_edge(hw_pad, s12p, src2d, dst2d, nb)
    return _finalize(accum, c)


# speedup vs baseline: 1.0335x; 1.0335x over previous
import functools

import jax
import jax.numpy as jnp
from jax import lax
from jax.experimental import pallas as pl
from jax.experimental.pallas import tpu as pltpu
from jax.experimental.pallas import tpu_sc as plsc

ALPHA = 0.2
LANES = 16
CP = 48
NC = 2
NS = 16
NW = NC * NS
CH = 128

_DN = lax.GatherDimensionNumbers(
    offset_dims=(), collapsed_slice_dims=(0,), start_index_map=(0,))


def _take16(vec, i):
    idx = jnp.full((LANES, 1), i, jnp.int32)
    return lax.gather(vec, idx, _DN, (1,),
                      mode=lax.GatherScatterMode.PROMISE_IN_BOUNDS)


def _project_body(h_ref, w_ref, a2_ref, hw_ref, s_ref):
    hW = jnp.dot(h_ref[...], w_ref[...], preferred_element_type=jnp.float32)
    b = hW.shape[0]
    ones = jnp.ones((b, 1), jnp.float32)
    zeros = jnp.zeros((b, CP - hW.shape[1] - 1), jnp.float32)
    hw_ref[...] = jnp.concatenate([hW, ones, zeros], axis=1)
    s_ref[...] = lax.dot_general(
        hW, a2_ref[...], (((1,), (1,)), ((), ())),
        preferred_element_type=jnp.float32)


def _project(h, W, a2):
    n, d = h.shape
    c = W.shape[1]
    blk = 2000
    grid = n // blk
    return pl.pallas_call(
        _project_body,
        grid=(grid,),
        in_specs=[
            pl.BlockSpec((blk, d), lambda i: (i, 0)),
            pl.BlockSpec((d, c), lambda i: (0, 0)),
            pl.BlockSpec((2, c), lambda i: (0, 0)),
        ],
        out_specs=[
            pl.BlockSpec((blk, CP), lambda i: (i, 0)),
            pl.BlockSpec((blk, 2), lambda i: (i, 0)),
        ],
        out_shape=[
            jax.ShapeDtypeStruct((n, CP), jnp.float32),
            jax.ShapeDtypeStruct((n, 2), jnp.float32),
        ],
    )(h, W, a2)


def _sc_edge(hw_pad, s12p, src2d, dst2d, nb):
    n = hw_pad.shape[0]
    s12n = s12p.shape[0]
    rpt = 640
    last_rows = n - (NS - 1) * rpt
    rpt_s = 1256
    last_s = s12n - (NS - 1) * rpt_s

    @functools.partial(
        pl.kernel,
        out_type=jax.ShapeDtypeStruct((NC, n, CP), jnp.float32),
        mesh=plsc.VectorSubcoreMesh(core_axis_name="c", subcore_axis_name="s"),
        compiler_params=pltpu.CompilerParams(needs_layout_passes=False, use_tc_tiling_on_sc=False),
        scratch_types=[
            pltpu.VMEM((s12n,), jnp.float32),
            pltpu.VMEM_SHARED((s12n,), jnp.float32),
            pltpu.VMEM((nb, CH), jnp.int32),
            pltpu.VMEM((nb, CH), jnp.int32),
            pltpu.VMEM((CH, CP), jnp.float32),
            pltpu.VMEM((CH, CP), jnp.float32),
            pltpu.VMEM((CH,), jnp.float32),
            pltpu.VMEM((CH,), jnp.float32),
            pltpu.VMEM_SHARED((n + 8, CP), jnp.float32),
            pltpu.VMEM_SHARED((n, CP), jnp.float32),
            pltpu.SemaphoreType.DMA,
            pltpu.SemaphoreType.DMA,
            pltpu.SemaphoreType.DMA,
            pltpu.SemaphoreType.DMA,
        ],
    )
    def k(hw_hbm, s12_hbm, src_hbm, dst_hbm, out_hbm,
          s12_v, s12_sh, src_v, dst_v, rows_a, rows_b, w_a, w_b,
          acc_s, hw_s, g_a, g_b, s_a, s_b):
        sid = lax.axis_index("s")
        cid = lax.axis_index("c")
        wid = sid * NC + cid

        @pl.when(sid < NS - 1)
        def _():
            pltpu.sync_copy(hw_hbm.at[pl.ds(sid * rpt, rpt)],
                            hw_s.at[pl.ds(sid * rpt, rpt)])
            pltpu.sync_copy(s12_hbm.at[pl.ds(sid * rpt_s, rpt_s)],
                            s12_sh.at[pl.ds(sid * rpt_s, rpt_s)])

        @pl.when(sid == NS - 1)
        def _():
            pltpu.sync_copy(hw_hbm.at[pl.ds((NS - 1) * rpt, last_rows)],
                            hw_s.at[pl.ds((NS - 1) * rpt, last_rows)])
            pltpu.sync_copy(s12_hbm.at[pl.ds((NS - 1) * rpt_s, last_s)],
                            s12_sh.at[pl.ds((NS - 1) * rpt_s, last_s)])

        zv = jnp.zeros((LANES,), jnp.float32)
        for r in range(CH):
            for cc in range(CP // LANES):
                rows_a[r, pl.ds(cc * LANES, LANES)] = zv

        def _zero(base, cnt):
            for b in range(cnt // CH):
                pltpu.sync_copy(rows_a, acc_s.at[pl.ds(base + b * CH, CH)])
            rem = cnt % CH
            if rem:
                pltpu.sync_copy(
                    rows_a.at[pl.ds(0, rem)],
                    acc_s.at[pl.ds(base + (cnt // CH) * CH, rem)])

        @pl.when(sid < NS - 1)
        def _():
            _zero(sid * rpt, rpt)

        @pl.when(sid == NS - 1)
        def _():
            _zero((NS - 1) * rpt, last_rows)

        pltpu.sync_copy(src_hbm.at[pl.ds(wid * nb, nb)], src_v)
        pltpu.sync_copy(dst_hbm.at[pl.ds(wid * nb, nb)], dst_v)

        plsc.subcore_barrier()

        pltpu.sync_copy(s12_sh, s12_v)

        def weights(j):
            ws = []
            for g in range(CH // LANES):
                srcv = src_v[j, pl.ds(g * LANES, LANES)]
                dstv = dst_v[j, pl.ds(g * LANES, LANES)]
                s1 = plsc.load_gather(s12_v, [srcv * 2])
                s2 = plsc.load_gather(s12_v, [dstv * 2 + 1])
                logit = s1 + s2
                lk = jnp.where(logit >= 0.0, logit, logit * ALPHA)
                ws.append(jnp.exp(-lk))
            return ws

        def scale(rows_v, ws):
            for g in range(CH // LANES):
                for i in range(LANES):
                    wi = _take16(ws[g], i)
                    r = g * LANES + i
                    for cc in range(CP // LANES):
                        sl = pl.ds(cc * LANES, LANES)
                        rows_v[r, sl] = rows_v[r, sl] * wi

        def gather(j, buf, sem):
            pltpu.async_copy(hw_s.at[dst_v.at[j]], buf, sem)

        def gather_wait(buf, sem):
            pltpu.make_async_copy(hw_s.at[dst_v.at[0]], buf, sem).wait()

        def scat(j, buf, sem):
            pltpu.async_copy(buf, acc_s.at[src_v.at[j]], sem, add=True)

        def scat_wait(buf, sem):
            pltpu.make_async_copy(buf, acc_s.at[src_v.at[0]], sem).wait()

        def body(i, carry):
            t = 2 * i
            gather(t, rows_a, g_a)
            ws_a = weights(t)
            gather_wait(rows_a, g_a)
            gather(t + 1, rows_b, g_b)
            scale(rows_a, ws_a)
            scat(t, rows_a, s_a)
            ws_b = weights(t + 1)
            gather_wait(rows_b, g_b)
            scale(rows_b, ws_b)
            scat(t + 1, rows_b, s_b)
            scat_wait(rows_a, s_a)
            scat_wait(rows_b, s_b)
            return carry

        lax.fori_loop(0, nb // 2, body, 0)

        plsc.subcore_barrier()

        @pl.when(sid < NS - 1)
        def _():
            pltpu.sync_copy(acc_s.at[pl.ds(sid * rpt, rpt)],
                            out_hbm.at[cid, pl.ds(sid * rpt, rpt)])

        @pl.when(sid == NS - 1)
        def _():
            pltpu.sync_copy(
                acc_s.at[pl.ds((NS - 1) * rpt, last_rows)],
                out_hbm.at[cid, pl.ds((NS - 1) * rpt, last_rows)])

    return k(hw_pad, s12p, src2d, dst2d)


def _finalize_body(acc_ref, out_ref):
    a0 = acc_ref[0]
    a1 = acc_ref[1]
    c = out_ref.shape[1]
    num = a0[:, :c] + a1[:, :c]
    den = a0[:, c:c + 1] + a1[:, c:c + 1]
    hp = num / den
    out_ref[...] = jnp.where(hp > 0.0, hp, jnp.exp(hp) - 1.0)


def _finalize(accum, c):
    n = accum.shape[1]
    blk = 1000
    grid = n // blk
    return pl.pallas_call(
        _finalize_body,
        grid=(grid,),
        in_specs=[pl.BlockSpec((NC, blk, CP), lambda i: (0, i, 0))],
        out_specs=pl.BlockSpec((blk, c), lambda i: (i, 0)),
        out_shape=jax.ShapeDtypeStruct((n, c), jnp.float32),
    )(accum)


def kernel(h, edge_index, W, a):
    n = h.shape[0]
    c = W.shape[1]
    e = edge_index.shape[1]
    hw_pad, s12 = _project(h, W, a.reshape(2, c))
    nch = e // CH
    nb = -(-nch // NW)
    nb = nb + (nb % 2)
    pad = nb * NW - nch
    src2d = jnp.pad(edge_index[0], (0, pad * CH),
                    constant_values=n).reshape(nb * NW, CH)
    dst2d = jnp.pad(edge_index[1], (0, pad * CH)).reshape(nb * NW, CH)
    s12p = jnp.pad(s12.reshape(2 * n), (0, 16))
    accum = _sc_edge(hw_pad, s12p, src2d, dst2d, nb)
    return _finalize(accum, c)


# speedup vs baseline: 1.0629x; 1.0284x over previous
import functools

import jax
import jax.numpy as jnp
from jax import lax
from jax.experimental import pallas as pl
from jax.experimental.pallas import tpu as pltpu
from jax.experimental.pallas import tpu_sc as plsc

ALPHA = 0.2
LANES = 16
CP = 48
NC = 2
NS = 16
NW = NC * NS
CH = 128

_DN = lax.GatherDimensionNumbers(
    offset_dims=(), collapsed_slice_dims=(0,), start_index_map=(0,))


def _take16(vec, i):
    idx = jnp.full((LANES, 1), i, jnp.int32)
    return lax.gather(vec, idx, _DN, (1,),
                      mode=lax.GatherScatterMode.PROMISE_IN_BOUNDS)


def _project_body(h_ref, w_ref, a2_ref, hw_ref, s_ref):
    hW = jnp.dot(h_ref[...], w_ref[...], preferred_element_type=jnp.float32)
    b = hW.shape[0]
    ones = jnp.ones((b, 1), jnp.float32)
    zeros = jnp.zeros((b, CP - hW.shape[1] - 1), jnp.float32)
    hw_ref[...] = jnp.concatenate([hW, ones, zeros], axis=1)
    s_ref[...] = lax.dot_general(
        hW, a2_ref[...], (((1,), (1,)), ((), ())),
        preferred_element_type=jnp.float32)


def _project(h, W, a2):
    n, d = h.shape
    c = W.shape[1]
    blk = 2000
    grid = n // blk
    return pl.pallas_call(
        _project_body,
        grid=(grid,),
        in_specs=[
            pl.BlockSpec((blk, d), lambda i: (i, 0)),
            pl.BlockSpec((d, c), lambda i: (0, 0)),
            pl.BlockSpec((2, c), lambda i: (0, 0)),
        ],
        out_specs=[
            pl.BlockSpec((blk, CP), lambda i: (i, 0)),
            pl.BlockSpec((blk, 2), lambda i: (i, 0)),
        ],
        out_shape=[
            jax.ShapeDtypeStruct((n, CP), jnp.float32),
            jax.ShapeDtypeStruct((n, 2), jnp.float32),
        ],
    )(h, W, a2)


def _sc_edge(hw_pad, s12p, src2d, dst2d, nb):
    n = hw_pad.shape[0]
    s12n = s12p.shape[0]
    rpt = 640
    last_rows = n - (NS - 1) * rpt
    rpt_s = 1256
    last_s = s12n - (NS - 1) * rpt_s

    @functools.partial(
        pl.kernel,
        out_type=jax.ShapeDtypeStruct((NC, n, CP), jnp.float32),
        mesh=plsc.VectorSubcoreMesh(core_axis_name="c", subcore_axis_name="s"),
        compiler_params=pltpu.CompilerParams(needs_layout_passes=False, use_tc_tiling_on_sc=False),
        scratch_types=[
            pltpu.VMEM((s12n,), jnp.float32),
            pltpu.VMEM_SHARED((s12n,), jnp.float32),
            pltpu.VMEM((nb, CH), jnp.int32),
            pltpu.VMEM((nb, CH), jnp.int32),
            pltpu.VMEM((CH, CP), jnp.float32),
            pltpu.VMEM((CH, CP), jnp.float32),
            pltpu.VMEM((CH,), jnp.float32),
            pltpu.VMEM((CH,), jnp.float32),
            pltpu.VMEM_SHARED((n + 8, CP), jnp.float32),
            pltpu.VMEM_SHARED((n, CP), jnp.float32),
            pltpu.SemaphoreType.DMA,
            pltpu.SemaphoreType.DMA,
            pltpu.SemaphoreType.DMA,
            pltpu.SemaphoreType.DMA,
        ],
    )
    def k(hw_hbm, s12_hbm, src_hbm, dst_hbm, out_hbm,
          s12_v, s12_sh, src_v, dst_v, rows_a, rows_b, w_a, w_b,
          acc_s, hw_s, g_a, g_b, s_a, s_b):
        sid = lax.axis_index("s")
        cid = lax.axis_index("c")
        wid = sid * NC + cid

        @pl.when(sid < NS - 1)
        def _():
            pltpu.async_copy(hw_hbm.at[pl.ds(sid * rpt, rpt)],
                             hw_s.at[pl.ds(sid * rpt, rpt)], g_a)
            pltpu.async_copy(s12_hbm.at[pl.ds(sid * rpt_s, rpt_s)],
                             s12_sh.at[pl.ds(sid * rpt_s, rpt_s)], g_b)

        @pl.when(sid == NS - 1)
        def _():
            pltpu.async_copy(hw_hbm.at[pl.ds((NS - 1) * rpt, last_rows)],
                             hw_s.at[pl.ds((NS - 1) * rpt, last_rows)], g_a)
            pltpu.async_copy(s12_hbm.at[pl.ds((NS - 1) * rpt_s, last_s)],
                             s12_sh.at[pl.ds((NS - 1) * rpt_s, last_s)], g_b)

        zv = jnp.zeros((LANES,), jnp.float32)
        for r in range(CH):
            for cc in range(CP // LANES):
                rows_a[r, pl.ds(cc * LANES, LANES)] = zv

        def _zero(base, cnt):
            for b in range(cnt // CH):
                pltpu.sync_copy(rows_a, acc_s.at[pl.ds(base + b * CH, CH)])
            rem = cnt % CH
            if rem:
                pltpu.sync_copy(
                    rows_a.at[pl.ds(0, rem)],
                    acc_s.at[pl.ds(base + (cnt // CH) * CH, rem)])

        @pl.when(sid < NS - 1)
        def _():
            _zero(sid * rpt, rpt)

        @pl.when(sid == NS - 1)
        def _():
            _zero((NS - 1) * rpt, last_rows)

        pltpu.async_copy(src_hbm.at[pl.ds(wid * nb, nb)], src_v, s_a)
        pltpu.async_copy(dst_hbm.at[pl.ds(wid * nb, nb)], dst_v, s_b)

        @pl.when(sid < NS - 1)
        def _():
            pltpu.make_async_copy(hw_hbm.at[pl.ds(sid * rpt, rpt)],
                                  hw_s.at[pl.ds(sid * rpt, rpt)], g_a).wait()
            pltpu.make_async_copy(
                s12_hbm.at[pl.ds(sid * rpt_s, rpt_s)],
                s12_sh.at[pl.ds(sid * rpt_s, rpt_s)], g_b).wait()

        @pl.when(sid == NS - 1)
        def _():
            pltpu.make_async_copy(
                hw_hbm.at[pl.ds((NS - 1) * rpt, last_rows)],
                hw_s.at[pl.ds((NS - 1) * rpt, last_rows)], g_a).wait()
            pltpu.make_async_copy(
                s12_hbm.at[pl.ds((NS - 1) * rpt_s, last_s)],
                s12_sh.at[pl.ds((NS - 1) * rpt_s, last_s)], g_b).wait()

        pltpu.make_async_copy(src_hbm.at[pl.ds(wid * nb, nb)], src_v,
                              s_a).wait()
        pltpu.make_async_copy(dst_hbm.at[pl.ds(wid * nb, nb)], dst_v,
                              s_b).wait()

        plsc.subcore_barrier()

        pltpu.sync_copy(s12_sh, s12_v)

        def weights(j):
            ws = []
            for g in range(CH // LANES):
                srcv = src_v[j, pl.ds(g * LANES, LANES)]
                dstv = dst_v[j, pl.ds(g * LANES, LANES)]
                s1 = plsc.load_gather(s12_v, [srcv * 2])
                s2 = plsc.load_gather(s12_v, [dstv * 2 + 1])
                logit = s1 + s2
                lk = jnp.where(logit >= 0.0, logit, logit * ALPHA)
                ws.append(jnp.exp(-lk))
            return ws

        def scale(rows_v, ws):
            for g in range(CH // LANES):
                for i in range(LANES):
                    wi = _take16(ws[g], i)
                    r = g * LANES + i
                    for cc in range(CP // LANES):
                        sl = pl.ds(cc * LANES, LANES)
                        rows_v[r, sl] = rows_v[r, sl] * wi

        def gather(j, buf, sem):
            pltpu.async_copy(hw_s.at[dst_v.at[j]], buf, sem)

        def gather_wait(buf, sem):
            pltpu.make_async_copy(hw_s.at[dst_v.at[0]], buf, sem).wait()

        def scat(j, buf, sem):
            pltpu.async_copy(buf, acc_s.at[src_v.at[j]], sem, add=True)

        def scat_wait(buf, sem):
            pltpu.make_async_copy(buf, acc_s.at[src_v.at[0]], sem).wait()

        def body(i, carry):
            t = 2 * i
            gather(t, rows_a, g_a)
            ws_a = weights(t)
            gather_wait(rows_a, g_a)
            gather(t + 1, rows_b, g_b)
            scale(rows_a, ws_a)
            scat(t, rows_a, s_a)
            ws_b = weights(t + 1)
            gather_wait(rows_b, g_b)
            scale(rows_b, ws_b)
            scat(t + 1, rows_b, s_b)
            scat_wait(rows_a, s_a)
            scat_wait(rows_b, s_b)
            return carry

        lax.fori_loop(0, nb // 2, body, 0)

        plsc.subcore_barrier()

        @pl.when(sid < NS - 1)
        def _():
            pltpu.sync_copy(acc_s.at[pl.ds(sid * rpt, rpt)],
                            out_hbm.at[cid, pl.ds(sid * rpt, rpt)])

        @pl.when(sid == NS - 1)
        def _():
            pltpu.sync_copy(
                acc_s.at[pl.ds((NS - 1) * rpt, last_rows)],
                out_hbm.at[cid, pl.ds((NS - 1) * rpt, last_rows)])

    return k(hw_pad, s12p, src2d, dst2d)


def _finalize_body(acc_ref, out_ref):
    a0 = acc_ref[0]
    a1 = acc_ref[1]
    c = out_ref.shape[1]
    num = a0[:, :c] + a1[:, :c]
    den = a0[:, c:c + 1] + a1[:, c:c + 1]
    hp = num / den
    out_ref[...] = jnp.where(hp > 0.0, hp, jnp.exp(hp) - 1.0)


def _finalize(accum, c):
    n = accum.shape[1]
    blk = 1000
    grid = n // blk
    return pl.pallas_call(
        _finalize_body,
        grid=(grid,),
        in_specs=[pl.BlockSpec((NC, blk, CP), lambda i: (0, i, 0))],
        out_specs=pl.BlockSpec((blk, c), lambda i: (i, 0)),
        out_shape=jax.ShapeDtypeStruct((n, c), jnp.float32),
    )(accum)


def kernel(h, edge_index, W, a):
    n = h.shape[0]
    c = W.shape[1]
    e = edge_index.shape[1]
    hw_pad, s12 = _project(h, W, a.reshape(2, c))
    nch = e // CH
    nb = -(-nch // NW)
    nb = nb + (nb % 2)
    pad = nb * NW - nch
    src2d = jnp.pad(edge_index[0], (0, pad * CH),
                    constant_values=n).reshape(nb * NW, CH)
    dst2d = jnp.pad(edge_index[1], (0, pad * CH)).reshape(nb * NW, CH)
    s12p = jnp.pad(s12.reshape(2 * n), (0, 16))
    accum = _sc_edge(hw_pad, s12p, src2d, dst2d, nb)
    return _finalize(accum, c)
